# Initial kernel scaffold; baseline (speedup 1.0000x reference)
import jax
import jax.numpy as jnp
from jax.experimental import pallas as pl


def kernel(X, tables):
    raise NotImplementedError("write your pallas kernel here")


# speedup vs baseline: 1.0235x; 1.0235x over previous
import functools

import jax
import jax.numpy as jnp
from jax import lax
from jax.experimental import pallas as pl
from jax.experimental.pallas import tpu as pltpu
from jax.experimental.pallas import tpu_sc as plsc

_N_FIELDS = 26
_VOCAB = 100000
_DIM = 64
_BATCH = 16384

_NC = 2
_NS = 16
_NW = _NC * _NS

_B_TOTAL = _N_FIELDS * _BATCH
_R = _B_TOTAL // _NW
_C = 128
_NCHUNK = _R // _C
_K = 8
_NGRP = _NCHUNK // _K
_GROWS = _K * _C

_mesh = plsc.VectorSubcoreMesh(core_axis_name="c", subcore_axis_name="s")


@functools.partial(
    pl.kernel,
    mesh=_mesh,
    compiler_params=pltpu.CompilerParams(use_tc_tiling_on_sc=False),
    out_type=jax.ShapeDtypeStruct((_B_TOTAL, _DIM), jnp.float32),
    scratch_types=[
        pltpu.VMEM((_NCHUNK, _C), jnp.int32),
        pltpu.VMEM((_GROWS, _DIM), jnp.float32),
        pltpu.SemaphoreType.DMA,
    ],
)
def _sc_gather(x_hbm, tables_hbm, out_hbm, idx_v, rows_v, gsem):
    wid = lax.axis_index("s") * _NC + lax.axis_index("c")
    base_row = wid * _R

    pltpu.sync_copy(x_hbm.at[wid], idx_v)

    def _add_offsets(c, carry):
        off = ((base_row + c * _C) // _BATCH) * _VOCAB
        for s in range(_C // 16):
            sl = pl.ds(s * 16, 16)
            idx_v[c, sl] = idx_v[c, sl] + off
        return carry

    lax.fori_loop(0, _NCHUNK, _add_offsets, 0)

    def _group(g, carry):
        copies = []
        for j in range(_K):
            c = g * _K + j
            copies.append(
                pltpu.async_copy(
                    tables_hbm.at[idx_v.at[c]],
                    rows_v.at[pl.ds(j * _C, _C)],
                    gsem,
                )
            )
        for cp in copies:
            cp.wait()
        pltpu.sync_copy(rows_v, out_hbm.at[pl.ds(base_row + g * _GROWS, _GROWS)])
        return carry

    lax.fori_loop(0, _NGRP, _group, 0)


def kernel(X, tables):
    x_resh = X.reshape(_NW, _NCHUNK, _C)
    tables_flat = tables.reshape(_N_FIELDS * _VOCAB, _DIM)
    return _sc_gather(x_resh, tables_flat)


# speedup vs baseline: 1.0238x; 1.0003x over previous
import functools

import jax
import jax.numpy as jnp
from jax import lax
from jax.experimental import pallas as pl
from jax.experimental.pallas import tpu as pltpu
from jax.experimental.pallas import tpu_sc as plsc

_N_FIELDS = 26
_VOCAB = 100000
_DIM = 64
_BATCH = 16384

_NC = 2
_NS = 16
_NW = _NC * _NS

_B_TOTAL = _N_FIELDS * _BATCH
_R = _B_TOTAL // _NW
_C = 128
_NCHUNK = _R // _C
_K = 4
_NGRP = _NCHUNK // _K
_GROWS = _K * _C

_mesh = plsc.VectorSubcoreMesh(core_axis_name="c", subcore_axis_name="s")


@functools.partial(
    pl.kernel,
    mesh=_mesh,
    compiler_params=pltpu.CompilerParams(use_tc_tiling_on_sc=False),
    out_type=jax.ShapeDtypeStruct((_B_TOTAL, _DIM), jnp.float32),
    scratch_types=[
        pltpu.VMEM((_NCHUNK, _C), jnp.int32),
        pltpu.VMEM((2 * _GROWS, _DIM), jnp.float32),
        pltpu.SemaphoreType.DMA,
        pltpu.SemaphoreType.DMA,
    ],
)
def _sc_gather(x_hbm, tables_hbm, out_hbm, idx_v, rows_v, gsem, osem):
    wid = lax.axis_index("s") * _NC + lax.axis_index("c")
    base_row = wid * _R

    pltpu.sync_copy(x_hbm.at[wid], idx_v)

    def _add_offsets(c, carry):
        off = ((base_row + c * _C) // _BATCH) * _VOCAB
        for s in range(_C // 16):
            sl = pl.ds(s * 16, 16)
            idx_v[c, sl] = idx_v[c, sl] + off
        return carry

    lax.fori_loop(0, _NCHUNK, _add_offsets, 0)

    def _fire_gathers(g, b):
        for j in range(_K):
            pltpu.async_copy(
                tables_hbm.at[idx_v.at[g * _K + j]],
                rows_v.at[pl.ds(b * _GROWS + j * _C, _C)],
                gsem,
            )

    def _drain_gathers(b):
        for j in range(_K):
            pltpu.make_async_copy(
                tables_hbm.at[pl.ds(0, _C)],
                rows_v.at[pl.ds(b * _GROWS + j * _C, _C)],
                gsem,
            ).wait()

    def _fire_writeback(g, b):
        pltpu.async_copy(
            rows_v.at[pl.ds(b * _GROWS, _GROWS)],
            out_hbm.at[pl.ds(base_row + g * _GROWS, _GROWS)],
            osem,
        )

    def _drain_writeback(b):
        pltpu.make_async_copy(
            tables_hbm.at[pl.ds(0, _GROWS)],
            rows_v.at[pl.ds(b * _GROWS, _GROWS)],
            osem,
        ).wait()

    _fire_gathers(0, 0)
    _drain_gathers(0)
    _fire_writeback(0, 0)
    _fire_gathers(1, 1)

    def _group(g, carry):
        b = g % 2
        nb = 1 - b
        _drain_gathers(b)
        _drain_writeback(nb)
        _fire_gathers(g + 1, nb)
        _fire_writeback(g, b)
        return carry

    lax.fori_loop(1, _NGRP - 1, _group, 0)

    _drain_gathers(1)
    _drain_writeback(0)
    _fire_writeback(_NGRP - 1, 1)
    _drain_writeback(1)


def kernel(X, tables):
    x_resh = X.reshape(_NW, _NCHUNK, _C)
    tables_flat = tables.reshape(_N_FIELDS * _VOCAB, _DIM)
    return _sc_gather(x_resh, tables_flat)
